# baseline (speedup 1.0000x reference)
import functools

import jax
import jax.numpy as jnp
from jax import lax
from jax.experimental import pallas as pl
from jax.experimental.pallas import tpu as pltpu, tpu_sc as plsc

_DIM = 32
_N_EMBED = 8192
_ROWS = 8192
_R_BLK = 4096
_CHUNK = 2048
_INT_MAX = 2**31 - 1


def _argmin_body(f_ref, e_ref, fsq_ref, esq_ref, ind_ref):
    f = f_ref[...]
    fsq = fsq_ref[...]
    fb = f.astype(jnp.bfloat16)

    def chunk_min(c):
        e = e_ref[:, pl.ds(c * _CHUNK, _CHUNK)]
        eb = e.astype(jnp.bfloat16)
        mm2 = lax.dot_general(fb, eb + eb,
                              (((1,), (0,)), ((), ())),
                              preferred_element_type=jnp.float32)
        esq = esq_ref[:, pl.ds(c * _CHUNK, _CHUNK)]
        d = (fsq - mm2) + esq
        m = jnp.min(d, axis=1, keepdims=True)
        ids = (lax.broadcasted_iota(jnp.int32, (_R_BLK, _CHUNK), 1)
               + c * _CHUNK)
        i = jnp.min(jnp.where(d == m, ids, _INT_MAX), axis=1, keepdims=True)
        return m, i

    def fold(c, carry):
        acc, idx = carry
        m, i = chunk_min(c)
        better = m < acc
        acc = jnp.where(better,
                        m.astype(jnp.bfloat16).astype(jnp.float32), acc)
        idx = jnp.where(better, i, idx)
        return acc, idx

    carry = (jnp.full((_R_BLK, 1), jnp.inf, jnp.float32),
             jnp.zeros((_R_BLK, 1), jnp.int32))
    for c in range(_N_EMBED // _CHUNK):
        carry = fold(c, carry)
    ind_ref[...] = carry[1]


def _tc_argmin(flatten, embed, fsq, esq):
    return pl.pallas_call(
        _argmin_body,
        grid=(_ROWS // _R_BLK,),
        in_specs=[pl.BlockSpec((_R_BLK, _DIM), lambda i: (i, 0)),
                  pl.BlockSpec((_DIM, _N_EMBED), lambda i: (0, 0)),
                  pl.BlockSpec((_R_BLK, 1), lambda i: (i, 0)),
                  pl.BlockSpec((1, _N_EMBED), lambda i: (0, 0))],
        out_specs=pl.BlockSpec((_R_BLK, 1), lambda i: (i, 0)),
        out_shape=jax.ShapeDtypeStruct((_ROWS, 1), jnp.int32),
    )(flatten, embed, fsq, esq)


_SC_INFO = plsc.get_sparse_core_info()
_NW = _SC_INFO.num_cores * _SC_INFO.num_subcores
_B_W = _ROWS // _NW


def _make_sc_gather():
    mesh = plsc.VectorSubcoreMesh(core_axis_name="c", subcore_axis_name="s")

    @functools.partial(
        pl.kernel, mesh=mesh,
        out_type=[jax.ShapeDtypeStruct((_ROWS, _DIM), jnp.float32),
                  jax.ShapeDtypeStruct((_ROWS, _DIM), jnp.float32)],
        scratch_types=[pltpu.VMEM((_B_W // 2,), jnp.int32),
                       pltpu.VMEM((_B_W // 2,), jnp.int32),
                       pltpu.VMEM((_B_W // 2, 128), jnp.float32),
                       pltpu.VMEM((_B_W, _DIM), jnp.float32),
                       pltpu.VMEM((_B_W, _DIM), jnp.float32),
                       pltpu.VMEM((_B_W, _DIM), jnp.float32),
                       pltpu.SemaphoreType.DMA],
    )
    def sc_gather(table_hbm, idx_hbm, x_hbm, qst_hbm, diff_hbm,
                  idx_v0, idx_v1, rows_v, x_v, qst_v, diff_v, sem):
        wid = lax.axis_index("s") * _SC_INFO.num_cores + lax.axis_index("c")
        base = wid * _B_W
        half = _B_W // 2
        pltpu.sync_copy(idx_hbm.at[pl.ds(base, half)], idx_v0)
        pltpu.sync_copy(idx_hbm.at[pl.ds(base + half, half)], idx_v1)
        pltpu.sync_copy(x_hbm.at[pl.ds(base, _B_W)], x_v)
        for b, idx_v in ((0, idx_v0), (1, idx_v1)):
            pltpu.async_copy(table_hbm.at[idx_v], rows_v, sem).wait()

            def row(i, _):
                for h in (0, 16):
                    r = rows_v[i, pl.ds(h, 16)]
                    xx = x_v[b * half + i, pl.ds(h, 16)]
                    dd = r - xx
                    diff_v[b * half + i, pl.ds(h, 16)] = dd
                    qst_v[b * half + i, pl.ds(h, 16)] = xx + dd
                return 0

            lax.fori_loop(0, half, row, 0)
        pltpu.sync_copy(qst_v, qst_hbm.at[pl.ds(base, _B_W)])
        pltpu.sync_copy(diff_v, diff_hbm.at[pl.ds(base, _B_W)])

    return sc_gather


_sc_gather = _make_sc_gather()


def kernel(input, embed):
    x = jnp.transpose(input, (0, 2, 3, 1))
    flatten = x.reshape(-1, _DIM)
    fsq = jnp.sum(flatten ** 2, axis=1, keepdims=True)
    esq = jnp.sum(embed ** 2, axis=0, keepdims=True)
    ind2d = _tc_argmin(flatten, embed, fsq, esq)
    idx = ind2d.reshape(_ROWS)
    table = jnp.pad(jnp.transpose(embed), ((0, 0), (0, 96)))
    qst, diff = _sc_gather(table, idx, flatten)
    embed_ind = ind2d.reshape(8, 32, 32)
    quantize_out = jnp.transpose(qst.reshape(8, 32, 32, _DIM), (0, 3, 1, 2))
    diff_out = diff.reshape(8, 32, 32, _DIM)
    return quantize_out, diff_out, embed_ind


# speedup vs baseline: 1.2805x; 1.2805x over previous
import functools

import jax
import jax.numpy as jnp
from jax import lax
from jax.experimental import pallas as pl
from jax.experimental.pallas import tpu as pltpu, tpu_sc as plsc

_DIM = 32
_N_EMBED = 8192
_ROWS = 8192
_R_BLK = 2048
_CHUNK = 2048
_INT_MAX = 2**31 - 1


def _argmin_body(f_ref, e_ref, fsq_ref, esq_ref, ind_ref):
    f = f_ref[...]
    fsq = fsq_ref[...]
    fb = f.astype(jnp.bfloat16)

    def chunk_min(c):
        e = e_ref[:, pl.ds(c * _CHUNK, _CHUNK)]
        eb = e.astype(jnp.bfloat16)
        mm2 = lax.dot_general(fb, eb + eb,
                              (((1,), (0,)), ((), ())),
                              preferred_element_type=jnp.float32)
        esq = esq_ref[:, pl.ds(c * _CHUNK, _CHUNK)]
        d = (fsq - mm2) + esq
        m = jnp.min(d, axis=1, keepdims=True)
        ids = (lax.broadcasted_iota(jnp.int32, (_R_BLK, _CHUNK), 1)
               + c * _CHUNK)
        i = jnp.min(jnp.where(d == m, ids, _INT_MAX), axis=1, keepdims=True)
        return m, i

    def fold(c, carry):
        acc, idx = carry
        m, i = chunk_min(c)
        better = m < acc
        acc = jnp.where(better,
                        m.astype(jnp.bfloat16).astype(jnp.float32), acc)
        idx = jnp.where(better, i, idx)
        return acc, idx

    carry = (jnp.full((_R_BLK, 1), jnp.inf, jnp.float32),
             jnp.zeros((_R_BLK, 1), jnp.int32))
    for c in range(_N_EMBED // _CHUNK):
        carry = fold(c, carry)
    ind_ref[...] = carry[1]


def _tc_argmin(flatten, embed, fsq, esq):
    return pl.pallas_call(
        _argmin_body,
        grid=(_ROWS // _R_BLK,),
        in_specs=[pl.BlockSpec((_R_BLK, _DIM), lambda i: (i, 0)),
                  pl.BlockSpec((_DIM, _N_EMBED), lambda i: (0, 0)),
                  pl.BlockSpec((_R_BLK, 1), lambda i: (i, 0)),
                  pl.BlockSpec((1, _N_EMBED), lambda i: (0, 0))],
        out_specs=pl.BlockSpec((_R_BLK, 1), lambda i: (i, 0)),
        out_shape=jax.ShapeDtypeStruct((_ROWS, 1), jnp.int32),
    )(flatten, embed, fsq, esq)


_SC_INFO = plsc.get_sparse_core_info()
_NW = _SC_INFO.num_cores * _SC_INFO.num_subcores
_B_W = _ROWS // _NW


def _make_sc_gather():
    mesh = plsc.VectorSubcoreMesh(core_axis_name="c", subcore_axis_name="s")

    @functools.partial(
        pl.kernel, mesh=mesh,
        out_type=[jax.ShapeDtypeStruct((_ROWS, _DIM), jnp.float32),
                  jax.ShapeDtypeStruct((_ROWS, _DIM), jnp.float32)],
        scratch_types=[pltpu.VMEM((_B_W // 2,), jnp.int32),
                       pltpu.VMEM((_B_W // 2,), jnp.int32),
                       pltpu.VMEM((_B_W // 2, 128), jnp.float32),
                       pltpu.VMEM((_B_W, _DIM), jnp.float32),
                       pltpu.VMEM((_B_W, _DIM), jnp.float32),
                       pltpu.VMEM((_B_W, _DIM), jnp.float32),
                       pltpu.SemaphoreType.DMA],
    )
    def sc_gather(table_hbm, idx_hbm, x_hbm, qst_hbm, diff_hbm,
                  idx_v0, idx_v1, rows_v, x_v, qst_v, diff_v, sem):
        wid = lax.axis_index("s") * _SC_INFO.num_cores + lax.axis_index("c")
        base = wid * _B_W
        half = _B_W // 2
        pltpu.sync_copy(idx_hbm.at[pl.ds(base, half)], idx_v0)
        pltpu.sync_copy(idx_hbm.at[pl.ds(base + half, half)], idx_v1)
        pltpu.sync_copy(x_hbm.at[pl.ds(base, _B_W)], x_v)
        for b, idx_v in ((0, idx_v0), (1, idx_v1)):
            pltpu.async_copy(table_hbm.at[idx_v], rows_v, sem).wait()

            def row(i, _):
                for h in (0, 16):
                    r = rows_v[i, pl.ds(h, 16)]
                    xx = x_v[b * half + i, pl.ds(h, 16)]
                    dd = r - xx
                    diff_v[b * half + i, pl.ds(h, 16)] = dd
                    qst_v[b * half + i, pl.ds(h, 16)] = xx + dd
                return 0

            lax.fori_loop(0, half, row, 0)
        pltpu.sync_copy(qst_v, qst_hbm.at[pl.ds(base, _B_W)])
        pltpu.sync_copy(diff_v, diff_hbm.at[pl.ds(base, _B_W)])

    return sc_gather


_sc_gather = _make_sc_gather()


def kernel(input, embed):
    x = jnp.transpose(input, (0, 2, 3, 1))
    flatten = x.reshape(-1, _DIM)
    fsq = jnp.sum(flatten ** 2, axis=1, keepdims=True)
    esq = jnp.sum(embed ** 2, axis=0, keepdims=True)
    ind2d = _tc_argmin(flatten, embed, fsq, esq)
    idx = ind2d.reshape(_ROWS)
    table = jnp.pad(jnp.transpose(embed), ((0, 0), (0, 96)))
    qst, diff = _sc_gather(table, idx, flatten)
    embed_ind = ind2d.reshape(8, 32, 32)
    quantize_out = jnp.transpose(qst.reshape(8, 32, 32, _DIM), (0, 3, 1, 2))
    diff_out = diff.reshape(8, 32, 32, _DIM)
    return quantize_out, diff_out, embed_ind
